# Initial kernel scaffold; baseline (speedup 1.0000x reference)
import jax
import jax.numpy as jnp
from jax.experimental import pallas as pl


def kernel(x):
    raise NotImplementedError("write your pallas kernel here")


# speedup vs baseline: 16.4617x; 16.4617x over previous
import functools

import jax
import jax.numpy as jnp
from jax.experimental import pallas as pl
from jax.experimental.pallas import tpu as pltpu

_BQ = 256


def _knn_body(q_ref, k_ref, o_ref):
    qi = pl.program_id(0)
    q = q_ref[...]
    k = k_ref[...]
    n = k.shape[0]

    g = jax.lax.dot_general(
        q, k, (((1,), (1,)), ((), ())), preferred_element_type=jnp.float32)
    sq_q = jnp.sum(q * q, axis=1, keepdims=True)
    sq_k = jnp.sum(k * k, axis=1, keepdims=True).T
    d2 = sq_q + sq_k - 2.0 * g
    d2 = jnp.maximum(d2, 0.0)

    rows = jax.lax.broadcasted_iota(jnp.int32, d2.shape, 0) + qi * _BQ
    cols = jax.lax.broadcasted_iota(jnp.int32, d2.shape, 1)
    d2 = jnp.where(rows == cols, 0.0, d2)

    inf = jnp.float32(jnp.inf)
    m1 = jnp.min(d2, axis=1, keepdims=True)
    d2a = jnp.where(d2 <= m1, inf, d2)
    m2 = jnp.min(d2a, axis=1, keepdims=True)
    d2b = jnp.where(d2a <= m2, inf, d2a)
    m3 = jnp.min(d2b, axis=1, keepdims=True)

    w = jnp.where((d2 <= m3) & (rows != cols),
                  1e-16 / jnp.maximum(d2, 1e-16), 0.0)
    s = jnp.sum(w, axis=1, keepdims=True)
    num = jax.lax.dot_general(
        w, k, (((1,), (0,)), ((), ())), preferred_element_type=jnp.float32)
    o_ref[...] = (q + num) / (1.0 + s)


@jax.jit
def kernel(x):
    n, d = x.shape
    return pl.pallas_call(
        _knn_body,
        grid=(n // _BQ,),
        in_specs=[
            pl.BlockSpec((_BQ, d), lambda i: (i, 0)),
            pl.BlockSpec((n, d), lambda i: (0, 0)),
        ],
        out_specs=pl.BlockSpec((_BQ, d), lambda i: (i, 0)),
        out_shape=jax.ShapeDtypeStruct((n, d), x.dtype),
    )(x, x)
